# baseline (speedup 1.0000x reference)
import jax
import jax.numpy as jnp
from jax import lax
from jax.experimental import pallas as pl
from jax.experimental.pallas import tpu as pltpu

T = 128
D = 1024
FF = 512
E = 64
K = 8
A = T * K
B = 16
NB = E + A // B
P = NB * B


def _route(top_k_index, top_k_weights):
    e = top_k_index.reshape(-1).astype(jnp.int32)
    w = top_k_weights.reshape(-1)
    t = (jnp.arange(A, dtype=jnp.int32) // K)
    oh = (e[:, None] == jnp.arange(E, dtype=jnp.int32)[None, :])
    counts = jnp.sum(oh.astype(jnp.int32), axis=0)
    nb = (counts + B - 1) // B
    offb = jnp.concatenate([jnp.zeros((1,), jnp.int32),
                            jnp.cumsum(nb)[:-1].astype(jnp.int32)])
    rank = jnp.take_along_axis(jnp.cumsum(oh.astype(jnp.int32), axis=0),
                               e[:, None], axis=1)[:, 0] - 1
    dst = offb[e] * B + rank
    tok = jnp.zeros((P,), jnp.int32).at[dst].set(t)
    wv = jnp.zeros((P,), jnp.float32).at[dst].set(w)
    blk_e = jnp.sum((jnp.arange(NB, dtype=jnp.int32)[:, None]
                     >= offb[None, :]).astype(jnp.int32), axis=1) - 1
    return tok.reshape(NB, 1, B), wv.reshape(NB, 1, B), blk_e


def _moe_body(blk_e_ref, tok_ref, wv_ref, x_ref, w1_ref, w3_ref, w2_ref, out_ref):
    s = pl.program_id(0)
    tok_b = tok_ref[0, 0, :]
    wv_b = wv_ref[0, 0, :]
    x = x_ref[...]
    w1 = w1_ref[0]
    w3 = w3_ref[0]
    w2 = w2_ref[0]

    col = lax.broadcasted_iota(jnp.int32, (B, T), 1)
    g_oh = (tok_b[:, None] == col).astype(jnp.float32)
    xb = jnp.dot(g_oh, x, preferred_element_type=jnp.float32)

    g = lax.dot_general(xb, w1, (((1,), (1,)), ((), ())),
                        preferred_element_type=jnp.float32)
    u = lax.dot_general(xb, w3, (((1,), (1,)), ((), ())),
                        preferred_element_type=jnp.float32)
    h = jax.nn.gelu(g, approximate=True) * u
    y = lax.dot_general(h, w2, (((1,), (1,)), ((), ())),
                        preferred_element_type=jnp.float32)

    s_oh = g_oh * wv_b[:, None]
    contrib = lax.dot_general(s_oh, y, (((0,), (0,)), ((), ())),
                              preferred_element_type=jnp.float32)

    @pl.when(s == 0)
    def _():
        out_ref[...] = contrib

    @pl.when(s != 0)
    def _():
        out_ref[...] += contrib


def kernel(hidden_states, top_k_index, top_k_weights, w1_weight, w2_weight, w3_weight):
    tok, wv, blk_e = _route(top_k_index, top_k_weights)
    grid_spec = pltpu.PrefetchScalarGridSpec(
        num_scalar_prefetch=1,
        grid=(NB,),
        in_specs=[
            pl.BlockSpec((1, 1, B), lambda s, be: (s, 0, 0)),
            pl.BlockSpec((1, 1, B), lambda s, be: (s, 0, 0)),
            pl.BlockSpec((T, D), lambda s, be: (0, 0)),
            pl.BlockSpec((1, FF, D), lambda s, be: (be[s], 0, 0)),
            pl.BlockSpec((1, FF, D), lambda s, be: (be[s], 0, 0)),
            pl.BlockSpec((1, D, FF), lambda s, be: (be[s], 0, 0)),
        ],
        out_specs=pl.BlockSpec((T, D), lambda s, be: (0, 0)),
    )
    return pl.pallas_call(
        _moe_body,
        grid_spec=grid_spec,
        out_shape=jax.ShapeDtypeStruct((T, D), jnp.float32),
    )(blk_e, tok, wv, hidden_states, w1_weight, w3_weight, w2_weight)


# speedup vs baseline: 1.2490x; 1.2490x over previous
import jax
import jax.numpy as jnp
from jax import lax
from jax.experimental import pallas as pl
from jax.experimental.pallas import tpu as pltpu

T = 128
D = 1024
FF = 512
E = 64
K = 8
A = T * K
B = 16
NB = E + A // B
P = NB * B


def _route(top_k_index, top_k_weights):
    e = top_k_index.reshape(-1).astype(jnp.int32)
    w = top_k_weights.reshape(-1)
    t = (jnp.arange(A, dtype=jnp.int32) // K)
    oh = (e[:, None] == jnp.arange(E, dtype=jnp.int32)[None, :])
    counts = jnp.sum(oh.astype(jnp.int32), axis=0)
    nb = (counts + B - 1) // B
    offb = jnp.concatenate([jnp.zeros((1,), jnp.int32),
                            jnp.cumsum(nb)[:-1].astype(jnp.int32)])
    rank = jnp.take_along_axis(jnp.cumsum(oh.astype(jnp.int32), axis=0),
                               e[:, None], axis=1)[:, 0] - 1
    dst = offb[e] * B + rank
    tok = jnp.zeros((P,), jnp.int32).at[dst].set(t)
    wv = jnp.zeros((P,), jnp.float32).at[dst].set(w)
    return tok.reshape(NB, B), wv.reshape(NB, B), offb, nb


def _moe_body(offb_ref, nb_ref, tok_ref, wv_ref, x_ref, w1_ref, w3_ref, w2_ref,
              out_ref):
    ei = pl.program_id(0)
    x = x_ref[...]
    w1 = w1_ref[0]
    w3 = w3_ref[0]
    w2 = w2_ref[0]
    off = offb_ref[ei]
    nblk = nb_ref[ei]

    @pl.when(ei == 0)
    def _():
        out_ref[...] = jnp.zeros((T, D), jnp.float32)

    rows = lax.broadcasted_iota(jnp.int32, (T, B), 0)

    def blk_step(i, _):
        blk = off + i
        tok_row = tok_ref[pl.ds(blk, 1), :]
        wv_row = wv_ref[pl.ds(blk, 1), :]
        oht = (rows == jnp.broadcast_to(tok_row, (T, B))).astype(jnp.float32)
        xb = lax.dot_general(oht, x, (((0,), (0,)), ((), ())),
                             preferred_element_type=jnp.float32)
        g = lax.dot_general(xb, w1, (((1,), (1,)), ((), ())),
                            preferred_element_type=jnp.float32)
        u = lax.dot_general(xb, w3, (((1,), (1,)), ((), ())),
                            preferred_element_type=jnp.float32)
        h = jax.nn.gelu(g, approximate=True) * u
        y = lax.dot_general(h, w2, (((1,), (1,)), ((), ())),
                            preferred_element_type=jnp.float32)
        soht = oht * jnp.broadcast_to(wv_row, (T, B))
        contrib = lax.dot_general(soht, y, (((1,), (0,)), ((), ())),
                                  preferred_element_type=jnp.float32)
        out_ref[...] += contrib
        return 0

    lax.fori_loop(0, nblk, blk_step, 0)


def kernel(hidden_states, top_k_index, top_k_weights, w1_weight, w2_weight, w3_weight):
    tok, wv, offb, nb = _route(top_k_index, top_k_weights)
    grid_spec = pltpu.PrefetchScalarGridSpec(
        num_scalar_prefetch=2,
        grid=(E,),
        in_specs=[
            pl.BlockSpec((NB, B), lambda e, offb, nb: (0, 0)),
            pl.BlockSpec((NB, B), lambda e, offb, nb: (0, 0)),
            pl.BlockSpec((T, D), lambda e, offb, nb: (0, 0)),
            pl.BlockSpec((1, FF, D), lambda e, offb, nb: (e, 0, 0)),
            pl.BlockSpec((1, FF, D), lambda e, offb, nb: (e, 0, 0)),
            pl.BlockSpec((1, D, FF), lambda e, offb, nb: (e, 0, 0)),
        ],
        out_specs=pl.BlockSpec((T, D), lambda e, offb, nb: (0, 0)),
    )
    return pl.pallas_call(
        _moe_body,
        grid_spec=grid_spec,
        out_shape=jax.ShapeDtypeStruct((T, D), jnp.float32),
    )(offb, nb, tok, wv, hidden_states, w1_weight, w3_weight, w2_weight)


# speedup vs baseline: 2.2006x; 1.7619x over previous
import jax
import jax.numpy as jnp
from jax import lax
from jax.experimental import pallas as pl

T = 128
D = 1024
FF = 512
E = 64
K = 8


def _moe_body(idx_ref, wts_ref, x_ref, w1_ref, w3_ref, w2_ref, out_ref):
    e = pl.program_id(0)
    x = x_ref[...].astype(jnp.bfloat16)
    w1 = w1_ref[0].astype(jnp.bfloat16)
    w3 = w3_ref[0].astype(jnp.bfloat16)
    w2 = w2_ref[0].astype(jnp.bfloat16)

    g = lax.dot_general(x, w1, (((1,), (1,)), ((), ())),
                        preferred_element_type=jnp.float32)
    u = lax.dot_general(x, w3, (((1,), (1,)), ((), ())),
                        preferred_element_type=jnp.float32)
    h = (jax.nn.gelu(g, approximate=True) * u).astype(jnp.bfloat16)
    y = lax.dot_general(h, w2, (((1,), (1,)), ((), ())),
                        preferred_element_type=jnp.float32)

    idx = idx_ref[...]
    wts = wts_ref[...]
    coef = jnp.sum(jnp.where(idx == e, wts, 0.0), axis=1)
    contrib = coef[:, None] * y

    @pl.when(e == 0)
    def _():
        out_ref[...] = contrib

    @pl.when(e != 0)
    def _():
        out_ref[...] += contrib


def kernel(hidden_states, top_k_index, top_k_weights, w1_weight, w2_weight, w3_weight):
    top_k_index = top_k_index.astype(jnp.int32)
    return pl.pallas_call(
        _moe_body,
        grid=(E,),
        in_specs=[
            pl.BlockSpec((T, K), lambda e: (0, 0)),
            pl.BlockSpec((T, K), lambda e: (0, 0)),
            pl.BlockSpec((T, D), lambda e: (0, 0)),
            pl.BlockSpec((1, FF, D), lambda e: (e, 0, 0)),
            pl.BlockSpec((1, FF, D), lambda e: (e, 0, 0)),
            pl.BlockSpec((1, D, FF), lambda e: (e, 0, 0)),
        ],
        out_specs=pl.BlockSpec((T, D), lambda e: (0, 0)),
        out_shape=jax.ShapeDtypeStruct((T, D), jnp.float32),
    )(top_k_index, top_k_weights, hidden_states, w1_weight, w3_weight, w2_weight)


# speedup vs baseline: 2.5095x; 1.1404x over previous
import jax
import jax.numpy as jnp
from jax import lax
from jax.experimental import pallas as pl

T = 128
D = 1024
FF = 512
E = 64
K = 8
EB = 2
NS = E // EB


def _moe_body(idx_ref, wts_ref, x_ref, w1_ref, w3_ref, w2_ref, out_ref):
    s = pl.program_id(0)
    x = x_ref[...]
    idx = idx_ref[...]
    wts = wts_ref[...]

    acc = jnp.zeros((T, D), jnp.float32)
    for j in range(EB):
        e = s * EB + j
        w1 = w1_ref[j]
        w3 = w3_ref[j]
        w2 = w2_ref[j]
        g = lax.dot_general(x, w1, (((1,), (1,)), ((), ())),
                            preferred_element_type=jnp.float32)
        u = lax.dot_general(x, w3, (((1,), (1,)), ((), ())),
                            preferred_element_type=jnp.float32)
        h = jax.nn.gelu(g, approximate=True) * u
        y = lax.dot_general(h, w2, (((1,), (1,)), ((), ())),
                            preferred_element_type=jnp.float32)
        coef = jnp.sum(jnp.where(idx == e, wts, 0.0), axis=1)
        acc = acc + coef[:, None] * y

    @pl.when(s == 0)
    def _():
        out_ref[...] = acc

    @pl.when(s != 0)
    def _():
        out_ref[...] += acc


def kernel(hidden_states, top_k_index, top_k_weights, w1_weight, w2_weight, w3_weight):
    top_k_index = top_k_index.astype(jnp.int32)
    return pl.pallas_call(
        _moe_body,
        grid=(NS,),
        in_specs=[
            pl.BlockSpec((T, K), lambda s: (0, 0)),
            pl.BlockSpec((T, K), lambda s: (0, 0)),
            pl.BlockSpec((T, D), lambda s: (0, 0)),
            pl.BlockSpec((EB, FF, D), lambda s: (s, 0, 0)),
            pl.BlockSpec((EB, FF, D), lambda s: (s, 0, 0)),
            pl.BlockSpec((EB, D, FF), lambda s: (s, 0, 0)),
        ],
        out_specs=pl.BlockSpec((T, D), lambda s: (0, 0)),
        out_shape=jax.ShapeDtypeStruct((T, D), jnp.float32),
    )(top_k_index, top_k_weights, hidden_states, w1_weight, w3_weight, w2_weight)
